# baseline (speedup 1.0000x reference)
import functools

import jax
import jax.numpy as jnp
from jax import lax
from jax.experimental import pallas as pl
from jax.experimental.pallas import tpu as pltpu
from jax.experimental.pallas import tpu_sc as plsc

N = 10000
E = 320000
G = 64
NC = 2
NS = 16
L = 16
NW = NC * NS
EPT = E // NW
CH = 128
PAD = 240
EPTP = EPT + PAD
NCHUNK = EPTP // CH
NP = 10240
RPT = NP // NS
TS = 640

_f32 = jnp.float32


def _make_segsum(w, nbuf):
  mesh = plsc.VectorSubcoreMesh(
      core_axis_name="c", subcore_axis_name="s", num_cores=NC, num_subcores=NS)

  @functools.partial(
      pl.kernel,
      out_type=jax.ShapeDtypeStruct((NC, NP, w), _f32),
      mesh=mesh,
      compiler_params=pltpu.CompilerParams(use_tc_tiling_on_sc=False),
      scratch_types=[
          pltpu.VMEM((NCHUNK, CH), jnp.int32),
          pltpu.VMEM((NCHUNK, CH), jnp.int32),
          [pltpu.VMEM((CH, w), _f32)] * nbuf,
          pltpu.VMEM_SHARED((N, w), _f32),
          pltpu.VMEM_SHARED((NP, w), _f32),
          [pltpu.SemaphoreType.DMA] * nbuf,
      ],
  )
  def segsum(table_hbm, src_hbm, dst_hbm, out_hbm,
             src_v, dst_v, rows_v, table_sh, acc_sh, gsem):
    cid = lax.axis_index("c")
    sid = lax.axis_index("s")
    wid = cid * NS + sid

    pltpu.sync_copy(src_hbm.at[wid], src_v)
    pltpu.sync_copy(dst_hbm.at[wid], dst_v)

    toff = pl.multiple_of(sid * TS, 8)

    @pl.when(sid < NS - 1)
    def _stage_full():
      pltpu.sync_copy(table_hbm.at[pl.ds(toff, TS)],
                      table_sh.at[pl.ds(toff, TS)])

    @pl.when(sid == NS - 1)
    def _stage_tail():
      pltpu.sync_copy(table_hbm.at[pl.ds((NS - 1) * TS, N - (NS - 1) * TS)],
                      table_sh.at[pl.ds((NS - 1) * TS, N - (NS - 1) * TS)])

    def zbody(r4, carry):
      for r in range(4):
        for c in range(w // L):
          rows_v[0][r4 * 4 + r, pl.ds(c * L, L)] = jnp.zeros((L,), _f32)
      return carry
    lax.fori_loop(0, CH // 4, zbody, 0)
    zoff = pl.multiple_of(sid * RPT, 8)
    for k in range(RPT // CH):
      pltpu.sync_copy(rows_v[0], acc_sh.at[pl.ds(zoff + k * CH, CH)])
    plsc.subcore_barrier()

    for b in range(nbuf):
      pltpu.async_copy(table_sh.at[src_v.at[b]], rows_v[b], gsem[b])

    def body(t, carry):
      for b in range(nbuf):
        j = t * nbuf + b
        pltpu.make_async_copy(
            table_sh.at[src_v.at[j]], rows_v[b], gsem[b]).wait()
        pltpu.sync_copy(rows_v[b], acc_sh.at[dst_v.at[j]], add=True)

        @pl.when(j + nbuf < NCHUNK)
        def _prefetch():
          pltpu.async_copy(
              table_sh.at[src_v.at[j + nbuf]], rows_v[b], gsem[b])
      return carry
    lax.fori_loop(0, NCHUNK // nbuf, body, 0)
    plsc.subcore_barrier()

    pltpu.sync_copy(acc_sh.at[pl.ds(zoff, RPT)],
                    out_hbm.at[cid, pl.ds(zoff, RPT)])

  return segsum


_segsum = {16: _make_segsum(16, 8), 32: _make_segsum(32, 8)}


def _relu(x):
  return jnp.maximum(x, 0.0)


def _mm_body(x_ref, w_ref, o_ref):
  o_ref[...] = jnp.dot(x_ref[...], w_ref[...], preferred_element_type=_f32)


def _mm_bias_body(x_ref, w_ref, b_ref, o_ref):
  o_ref[...] = (jnp.dot(x_ref[...], w_ref[...], preferred_element_type=_f32)
                + b_ref[...])


def _combine1_body(p_ref, hroot_ref, o_ref):
  p = p_ref[...]
  o_ref[...] = _relu(p[0, :N] + p[1, :N] + hroot_ref[...])


def _combine_body(p_ref, wr_ref, hroot_ref, o_ref):
  p = p_ref[...]
  o_ref[...] = _relu(
      jnp.dot(p[0, :N] + p[1, :N], wr_ref[...], preferred_element_type=_f32)
      + hroot_ref[...])


def _combine_split_body(p_ref, wr_ref, hroot_ref, oa_ref, ob_ref):
  p = p_ref[...]
  h = _relu(
      jnp.dot(p[0, :N] + p[1, :N], wr_ref[...], preferred_element_type=_f32)
      + hroot_ref[...])
  half = h.shape[1] // 2
  oa_ref[...] = h[:, :half]
  ob_ref[...] = h[:, half:]


def _root2_body(xa_ref, xb_ref, w_ref, b_ref, o_ref):
  w = w_ref[...]
  half = w.shape[0] // 2
  o_ref[...] = (
      jnp.dot(xa_ref[...], w[:half], preferred_element_type=_f32)
      + jnp.dot(xb_ref[...], w[half:], preferred_element_type=_f32)
      + b_ref[...])


def _sds(shape):
  return jax.ShapeDtypeStruct(shape, _f32)


def _tc_rel(x, w):
  return pl.pallas_call(_mm_body, out_shape=_sds((N, w.shape[1])))(x, w)


def _tc_root(x, w, b):
  return pl.pallas_call(_mm_bias_body, out_shape=_sds((N, w.shape[1])))(
      x, w, b.reshape(1, -1))


def _tc_combine1(p, hroot):
  return pl.pallas_call(_combine1_body, out_shape=_sds((N, hroot.shape[1])))(
      p, hroot)


def _tc_combine(p, wr, hroot):
  return pl.pallas_call(_combine_body, out_shape=_sds((N, wr.shape[1])))(
      p, wr, hroot)


def _tc_combine_split(p, wr, hroot):
  half = wr.shape[1] // 2
  return pl.pallas_call(
      _combine_split_body,
      out_shape=(_sds((N, half)), _sds((N, half))))(p, wr, hroot)


def _tc_root2(xa, xb, w, b):
  return pl.pallas_call(_root2_body, out_shape=_sds((N, w.shape[1])))(
      xa, xb, w, b.reshape(1, -1))


def _tc_final_body(pa_ref, pb_ref, wr_ref, hroot_ref, l1w_ref, l1b_ref,
                   batch_ref, l2w_ref, l2b_ref, out_ref):
  pa = pa_ref[...]
  pb = pb_ref[...]
  wr = wr_ref[...]
  half = wr.shape[0] // 2
  h4 = _relu(
      jnp.dot(pa[0, :N] + pa[1, :N], wr[:half], preferred_element_type=_f32)
      + jnp.dot(pb[0, :N] + pb[1, :N], wr[half:], preferred_element_type=_f32)
      + hroot_ref[...])
  hl = jnp.dot(h4, l1w_ref[...], preferred_element_type=_f32) + l1b_ref[...]
  gid = lax.broadcasted_iota(jnp.int32, (G, N), 0)
  mask = (gid == batch_ref[...]).astype(_f32)
  sums = jnp.dot(mask, hl, preferred_element_type=_f32)
  counts = jnp.sum(mask, axis=1, keepdims=True)
  pooled = sums / jnp.maximum(counts, 1.0)
  out_ref[...] = (jnp.dot(pooled, l2w_ref[...], preferred_element_type=_f32)
                  + l2b_ref[...])


_tc_final = pl.pallas_call(_tc_final_body, out_shape=_sds((G, 1)))


def kernel(x, edge_index, batch, W1_rel, W1_root, b1, W2_rel, W2_root, b2,
           W3_rel, W3_root, b3, W4_rel, W4_root, b4, lin1_W, lin1_b,
           lin2_W, lin2_b):
  pad_src = jnp.broadcast_to((jnp.arange(PAD, dtype=jnp.int32) * 37) % N,
                             (NW, PAD))
  pad_dst = jnp.broadcast_to(N + jnp.arange(PAD, dtype=jnp.int32), (NW, PAD))
  src = jnp.concatenate(
      [edge_index[0].reshape(NW, EPT), pad_src], axis=1).reshape(NW, NCHUNK, CH)
  dst = jnp.concatenate(
      [edge_index[1].reshape(NW, EPT), pad_dst], axis=1).reshape(NW, NCHUNK, CH)

  hrel1 = _tc_rel(x, W1_rel)
  hroot1 = _tc_root(x, W1_root, b1)
  p = _segsum[16](hrel1, src, dst)
  h1 = _tc_combine1(p, hroot1)

  hroot2 = _tc_root(h1, W2_root, b2)
  p = _segsum[16](h1, src, dst)
  h2 = _tc_combine(p, W2_rel, hroot2)

  hroot3 = _tc_root(h2, W3_root, b3)
  p = _segsum[32](h2, src, dst)
  h3a, h3b = _tc_combine_split(p, W3_rel, hroot3)

  hroot4 = _tc_root2(h3a, h3b, W4_root, b4)
  pa = _segsum[32](h3a, src, dst)
  pb = _segsum[32](h3b, src, dst)
  out = _tc_final(pa, pb, W4_rel, hroot4, lin1_W, lin1_b.reshape(1, -1),
                  batch.reshape(1, -1), lin2_W, lin2_b.reshape(1, -1))
  return out.reshape(-1)


# speedup vs baseline: 1.1501x; 1.1501x over previous
import functools

import jax
import jax.numpy as jnp
from jax import lax
from jax.experimental import pallas as pl
from jax.experimental.pallas import tpu as pltpu
from jax.experimental.pallas import tpu_sc as plsc

N = 10000
E = 320000
G = 64
NC = 2
NS = 16
L = 16
NW = NC * NS
EPT = E // NW
CH = 128
PAD = 240
EPTP = EPT + PAD
NCHUNK = EPTP // CH
NP = 10240
RPT = NP // NS
TS = 640

_f32 = jnp.float32


def _make_segsum(w, nbuf):
  mesh = plsc.VectorSubcoreMesh(
      core_axis_name="c", subcore_axis_name="s", num_cores=NC, num_subcores=NS)

  @functools.partial(
      pl.kernel,
      out_type=jax.ShapeDtypeStruct((NC, NP, w), _f32),
      mesh=mesh,
      compiler_params=pltpu.CompilerParams(use_tc_tiling_on_sc=False),
      scratch_types=[
          pltpu.VMEM((NCHUNK, CH), jnp.int32),
          pltpu.VMEM((NCHUNK, CH), jnp.int32),
          [pltpu.VMEM((CH, w), _f32)] * nbuf,
          pltpu.VMEM_SHARED((NP, w), _f32),
          [pltpu.SemaphoreType.DMA] * nbuf,
      ],
  )
  def segsum(table_hbm, src_hbm, dst_hbm, out_hbm,
             src_v, dst_v, rows_v, acc_sh, gsem):
    cid = lax.axis_index("c")
    sid = lax.axis_index("s")
    wid = cid * NS + sid

    pltpu.sync_copy(src_hbm.at[wid], src_v)
    pltpu.sync_copy(dst_hbm.at[wid], dst_v)

    def zbody(r4, carry):
      for r in range(4):
        for c in range(w // L):
          rows_v[0][r4 * 4 + r, pl.ds(c * L, L)] = jnp.zeros((L,), _f32)
      return carry
    lax.fori_loop(0, CH // 4, zbody, 0)
    zoff = pl.multiple_of(sid * RPT, 8)
    for k in range(RPT // CH):
      pltpu.sync_copy(rows_v[0], acc_sh.at[pl.ds(zoff + k * CH, CH)])
    plsc.subcore_barrier()

    for b in range(nbuf):
      pltpu.async_copy(table_hbm.at[src_v.at[b]], rows_v[b], gsem[b])

    def body(t, carry):
      for b in range(nbuf):
        j = t * nbuf + b
        pltpu.make_async_copy(
            table_hbm.at[src_v.at[j]], rows_v[b], gsem[b]).wait()
        pltpu.sync_copy(rows_v[b], acc_sh.at[dst_v.at[j]], add=True)

        @pl.when(j + nbuf < NCHUNK)
        def _prefetch():
          pltpu.async_copy(
              table_hbm.at[src_v.at[j + nbuf]], rows_v[b], gsem[b])
      return carry
    lax.fori_loop(0, NCHUNK // nbuf, body, 0)
    plsc.subcore_barrier()

    pltpu.sync_copy(acc_sh.at[pl.ds(zoff, RPT)],
                    out_hbm.at[cid, pl.ds(zoff, RPT)])

  return segsum


_segsum = {16: _make_segsum(16, 4), 32: _make_segsum(32, 4),
           64: _make_segsum(64, 4)}


def _relu(x):
  return jnp.maximum(x, 0.0)


def _mm_body(x_ref, w_ref, o_ref):
  o_ref[...] = jnp.dot(x_ref[...], w_ref[...], preferred_element_type=_f32)


def _mm_bias_body(x_ref, w_ref, b_ref, o_ref):
  o_ref[...] = (jnp.dot(x_ref[...], w_ref[...], preferred_element_type=_f32)
                + b_ref[...])


def _combine1_body(p_ref, hroot_ref, o_ref):
  p = p_ref[...]
  o_ref[...] = _relu(p[0, :N] + p[1, :N] + hroot_ref[...])


def _combine_body(p_ref, wr_ref, hroot_ref, o_ref):
  p = p_ref[...]
  o_ref[...] = _relu(
      jnp.dot(p[0, :N] + p[1, :N], wr_ref[...], preferred_element_type=_f32)
      + hroot_ref[...])


def _combine_split_body(p_ref, wr_ref, hroot_ref, oa_ref, ob_ref):
  p = p_ref[...]
  h = _relu(
      jnp.dot(p[0, :N] + p[1, :N], wr_ref[...], preferred_element_type=_f32)
      + hroot_ref[...])
  half = h.shape[1] // 2
  oa_ref[...] = h[:, :half]
  ob_ref[...] = h[:, half:]


def _root2_body(xa_ref, xb_ref, w_ref, b_ref, o_ref):
  w = w_ref[...]
  half = w.shape[0] // 2
  o_ref[...] = (
      jnp.dot(xa_ref[...], w[:half], preferred_element_type=_f32)
      + jnp.dot(xb_ref[...], w[half:], preferred_element_type=_f32)
      + b_ref[...])


def _sds(shape):
  return jax.ShapeDtypeStruct(shape, _f32)


def _tc_rel(x, w):
  return pl.pallas_call(_mm_body, out_shape=_sds((N, w.shape[1])))(x, w)


def _tc_root(x, w, b):
  return pl.pallas_call(_mm_bias_body, out_shape=_sds((N, w.shape[1])))(
      x, w, b.reshape(1, -1))


def _tc_combine1(p, hroot):
  return pl.pallas_call(_combine1_body, out_shape=_sds((N, hroot.shape[1])))(
      p, hroot)


def _tc_combine(p, wr, hroot):
  return pl.pallas_call(_combine_body, out_shape=_sds((N, wr.shape[1])))(
      p, wr, hroot)


def _tc_combine_split(p, wr, hroot):
  half = wr.shape[1] // 2
  return pl.pallas_call(
      _combine_split_body,
      out_shape=(_sds((N, half)), _sds((N, half))))(p, wr, hroot)


def _tc_root2(xa, xb, w, b):
  return pl.pallas_call(_root2_body, out_shape=_sds((N, w.shape[1])))(
      xa, xb, w, b.reshape(1, -1))


def _tc_final_body(p_ref, wr_ref, hroot_ref, l1w_ref, l1b_ref,
                   batch_ref, l2w_ref, l2b_ref, out_ref):
  p = p_ref[...]
  h4 = _relu(
      jnp.dot(p[0, :N] + p[1, :N], wr_ref[...], preferred_element_type=_f32)
      + hroot_ref[...])
  hl = jnp.dot(h4, l1w_ref[...], preferred_element_type=_f32) + l1b_ref[...]
  gid = lax.broadcasted_iota(jnp.int32, (G, N), 0)
  mask = (gid == batch_ref[...]).astype(_f32)
  sums = jnp.dot(mask, hl, preferred_element_type=_f32)
  counts = jnp.sum(mask, axis=1, keepdims=True)
  pooled = sums / jnp.maximum(counts, 1.0)
  out_ref[...] = (jnp.dot(pooled, l2w_ref[...], preferred_element_type=_f32)
                  + l2b_ref[...])


_tc_final = pl.pallas_call(_tc_final_body, out_shape=_sds((G, 1)))


def kernel(x, edge_index, batch, W1_rel, W1_root, b1, W2_rel, W2_root, b2,
           W3_rel, W3_root, b3, W4_rel, W4_root, b4, lin1_W, lin1_b,
           lin2_W, lin2_b):
  pad_src = jnp.broadcast_to((jnp.arange(PAD, dtype=jnp.int32) * 37) % N,
                             (NW, PAD))
  pad_dst = jnp.broadcast_to(N + jnp.arange(PAD, dtype=jnp.int32), (NW, PAD))
  src = jnp.concatenate(
      [edge_index[0].reshape(NW, EPT), pad_src], axis=1).reshape(NW, NCHUNK, CH)
  dst = jnp.concatenate(
      [edge_index[1].reshape(NW, EPT), pad_dst], axis=1).reshape(NW, NCHUNK, CH)

  hrel1 = _tc_rel(x, W1_rel)
  hroot1 = _tc_root(x, W1_root, b1)
  p = _segsum[16](hrel1, src, dst)
  h1 = _tc_combine1(p, hroot1)

  hroot2 = _tc_root(h1, W2_root, b2)
  p = _segsum[16](h1, src, dst)
  h2 = _tc_combine(p, W2_rel, hroot2)

  hroot3 = _tc_root(h2, W3_root, b3)
  p = _segsum[32](h2, src, dst)
  h3 = _tc_combine(p, W3_rel, hroot3)

  hroot4 = _tc_root(h3, W4_root, b4)
  p = _segsum[64](h3, src, dst)
  out = _tc_final(p, W4_rel, hroot4, lin1_W, lin1_b.reshape(1, -1),
                  batch.reshape(1, -1), lin2_W, lin2_b.reshape(1, -1))
  return out.reshape(-1)


# speedup vs baseline: 1.2211x; 1.0617x over previous
import functools

import jax
import jax.numpy as jnp
from jax import lax
from jax.experimental import pallas as pl
from jax.experimental.pallas import tpu as pltpu
from jax.experimental.pallas import tpu_sc as plsc

N = 10000
E = 320000
G = 64
NC = 2
NS = 16
L = 16
NW = NC * NS
EPT = E // NW
CH = 128
PAD = 240
EPTP = EPT + PAD
NCHUNK = EPTP // CH
NP = 10240
RPT = NP // NS
TS = 640

_f32 = jnp.float32


def _make_segsum(w, nbuf):
  mesh = plsc.VectorSubcoreMesh(
      core_axis_name="c", subcore_axis_name="s", num_cores=NC, num_subcores=NS)

  @functools.partial(
      pl.kernel,
      out_type=jax.ShapeDtypeStruct((NC, NP, w), _f32),
      mesh=mesh,
      compiler_params=pltpu.CompilerParams(use_tc_tiling_on_sc=False),
      scratch_types=[
          pltpu.VMEM((NCHUNK, CH), jnp.int32),
          pltpu.VMEM((NCHUNK, CH), jnp.int32),
          [pltpu.VMEM((CH, w), _f32)] * nbuf,
          pltpu.VMEM_SHARED((NP, w), _f32),
          [pltpu.SemaphoreType.DMA] * nbuf,
      ],
  )
  def segsum(table_hbm, src_hbm, dst_hbm, out_hbm,
             src_v, dst_v, rows_v, acc_sh, gsem):
    cid = lax.axis_index("c")
    sid = lax.axis_index("s")
    wid = cid * NS + sid

    pltpu.sync_copy(src_hbm.at[wid], src_v)
    pltpu.sync_copy(dst_hbm.at[wid], dst_v)

    def zbody(r4, carry):
      for r in range(4):
        for c in range(w // L):
          rows_v[0][r4 * 4 + r, pl.ds(c * L, L)] = jnp.zeros((L,), _f32)
      return carry
    lax.fori_loop(0, CH // 4, zbody, 0)
    zoff = pl.multiple_of(sid * RPT, 8)
    for k in range(RPT // CH):
      pltpu.sync_copy(rows_v[0], acc_sh.at[pl.ds(zoff + k * CH, CH)])
    plsc.subcore_barrier()

    for b in range(nbuf):
      pltpu.async_copy(table_hbm.at[src_v.at[b]], rows_v[b], gsem[b])

    def body(t, carry):
      for b in range(nbuf):
        j = t * nbuf + b
        pltpu.make_async_copy(
            table_hbm.at[src_v.at[j]], rows_v[b], gsem[b]).wait()
        pltpu.sync_copy(rows_v[b], acc_sh.at[dst_v.at[j]], add=True)

        @pl.when(j + nbuf < NCHUNK)
        def _prefetch():
          pltpu.async_copy(
              table_hbm.at[src_v.at[j + nbuf]], rows_v[b], gsem[b])
      return carry
    lax.fori_loop(0, NCHUNK // nbuf, body, 0)
    plsc.subcore_barrier()

    pltpu.sync_copy(acc_sh.at[pl.ds(zoff, RPT)],
                    out_hbm.at[cid, pl.ds(zoff, RPT)])

  return segsum


_segsum = {16: _make_segsum(16, 8), 32: _make_segsum(32, 8),
           64: _make_segsum(64, 8)}


def _relu(x):
  return jnp.maximum(x, 0.0)


def _mm_body(x_ref, w_ref, o_ref):
  o_ref[...] = jnp.dot(x_ref[...], w_ref[...], preferred_element_type=_f32)


def _mm_bias_body(x_ref, w_ref, b_ref, o_ref):
  o_ref[...] = (jnp.dot(x_ref[...], w_ref[...], preferred_element_type=_f32)
                + b_ref[...])


def _combine1_body(p_ref, hroot_ref, o_ref):
  p = p_ref[...]
  o_ref[...] = _relu(p[0, :N] + p[1, :N] + hroot_ref[...])


def _combine_body(p_ref, wr_ref, hroot_ref, o_ref):
  p = p_ref[...]
  o_ref[...] = _relu(
      jnp.dot(p[0, :N] + p[1, :N], wr_ref[...], preferred_element_type=_f32)
      + hroot_ref[...])


def _combine_split_body(p_ref, wr_ref, hroot_ref, oa_ref, ob_ref):
  p = p_ref[...]
  h = _relu(
      jnp.dot(p[0, :N] + p[1, :N], wr_ref[...], preferred_element_type=_f32)
      + hroot_ref[...])
  half = h.shape[1] // 2
  oa_ref[...] = h[:, :half]
  ob_ref[...] = h[:, half:]


def _root2_body(xa_ref, xb_ref, w_ref, b_ref, o_ref):
  w = w_ref[...]
  half = w.shape[0] // 2
  o_ref[...] = (
      jnp.dot(xa_ref[...], w[:half], preferred_element_type=_f32)
      + jnp.dot(xb_ref[...], w[half:], preferred_element_type=_f32)
      + b_ref[...])


def _sds(shape):
  return jax.ShapeDtypeStruct(shape, _f32)


def _tc_rel(x, w):
  return pl.pallas_call(_mm_body, out_shape=_sds((N, w.shape[1])))(x, w)


def _tc_root(x, w, b):
  return pl.pallas_call(_mm_bias_body, out_shape=_sds((N, w.shape[1])))(
      x, w, b.reshape(1, -1))


def _tc_combine1(p, hroot):
  return pl.pallas_call(_combine1_body, out_shape=_sds((N, hroot.shape[1])))(
      p, hroot)


def _tc_combine(p, wr, hroot):
  return pl.pallas_call(_combine_body, out_shape=_sds((N, wr.shape[1])))(
      p, wr, hroot)


def _tc_combine_split(p, wr, hroot):
  half = wr.shape[1] // 2
  return pl.pallas_call(
      _combine_split_body,
      out_shape=(_sds((N, half)), _sds((N, half))))(p, wr, hroot)


def _tc_root2(xa, xb, w, b):
  return pl.pallas_call(_root2_body, out_shape=_sds((N, w.shape[1])))(
      xa, xb, w, b.reshape(1, -1))


def _tc_final_body(p_ref, wr_ref, hroot_ref, l1w_ref, l1b_ref,
                   batch_ref, l2w_ref, l2b_ref, out_ref):
  p = p_ref[...]
  h4 = _relu(
      jnp.dot(p[0, :N] + p[1, :N], wr_ref[...], preferred_element_type=_f32)
      + hroot_ref[...])
  hl = jnp.dot(h4, l1w_ref[...], preferred_element_type=_f32) + l1b_ref[...]
  gid = lax.broadcasted_iota(jnp.int32, (G, N), 0)
  mask = (gid == batch_ref[...]).astype(_f32)
  sums = jnp.dot(mask, hl, preferred_element_type=_f32)
  counts = jnp.sum(mask, axis=1, keepdims=True)
  pooled = sums / jnp.maximum(counts, 1.0)
  out_ref[...] = (jnp.dot(pooled, l2w_ref[...], preferred_element_type=_f32)
                  + l2b_ref[...])


_tc_final = pl.pallas_call(_tc_final_body, out_shape=_sds((G, 1)))


def kernel(x, edge_index, batch, W1_rel, W1_root, b1, W2_rel, W2_root, b2,
           W3_rel, W3_root, b3, W4_rel, W4_root, b4, lin1_W, lin1_b,
           lin2_W, lin2_b):
  pad_src = jnp.broadcast_to((jnp.arange(PAD, dtype=jnp.int32) * 37) % N,
                             (NW, PAD))
  pad_dst = jnp.broadcast_to(N + jnp.arange(PAD, dtype=jnp.int32), (NW, PAD))
  src = jnp.concatenate(
      [edge_index[0].reshape(NW, EPT), pad_src], axis=1).reshape(NW, NCHUNK, CH)
  dst = jnp.concatenate(
      [edge_index[1].reshape(NW, EPT), pad_dst], axis=1).reshape(NW, NCHUNK, CH)

  hrel1 = _tc_rel(x, W1_rel)
  hroot1 = _tc_root(x, W1_root, b1)
  p = _segsum[16](hrel1, src, dst)
  h1 = _tc_combine1(p, hroot1)

  hroot2 = _tc_root(h1, W2_root, b2)
  p = _segsum[16](h1, src, dst)
  h2 = _tc_combine(p, W2_rel, hroot2)

  hroot3 = _tc_root(h2, W3_root, b3)
  p = _segsum[32](h2, src, dst)
  h3 = _tc_combine(p, W3_rel, hroot3)

  hroot4 = _tc_root(h3, W4_root, b4)
  p = _segsum[64](h3, src, dst)
  out = _tc_final(p, W4_rel, hroot4, lin1_W, lin1_b.reshape(1, -1),
                  batch.reshape(1, -1), lin2_W, lin2_b.reshape(1, -1))
  return out.reshape(-1)
